# baseline (speedup 1.0000x reference)
import functools

import jax
import jax.numpy as jnp
from jax import lax
from jax.experimental import pallas as pl
from jax.experimental.pallas import tpu as pltpu
from jax.experimental.pallas import tpu_sc as plsc

_info = plsc.get_sparse_core_info()
_NC, _NS = _info.num_cores, _info.num_subcores
_NW = _NC * _NS

_X_FLAT = 10000 * 128
_REL_FLAT = 500 * 128
_X_PER_W = _X_FLAT // _NW
_REL_PER_W = _REL_FLAT // _NW


def _make_sc_copy():
    mesh = plsc.VectorSubcoreMesh(core_axis_name="c", subcore_axis_name="s")

    @functools.partial(
        pl.kernel,
        mesh=mesh,
        out_type=[
            jax.ShapeDtypeStruct((_X_FLAT,), jnp.float32),
            jax.ShapeDtypeStruct((_REL_FLAT,), jnp.float32),
        ],
        scratch_types=[
            pltpu.VMEM((_X_PER_W,), jnp.float32),
            pltpu.VMEM((_REL_PER_W,), jnp.float32),
        ],
    )
    def sc_copy(x_hbm, rel_hbm, x_out_hbm, rel_out_hbm, xbuf, relbuf):
        wid = lax.axis_index("s") * _NC + lax.axis_index("c")
        xbase = wid * _X_PER_W
        rbase = wid * _REL_PER_W
        pltpu.sync_copy(x_hbm.at[pl.ds(xbase, _X_PER_W)], xbuf)
        pltpu.sync_copy(xbuf, x_out_hbm.at[pl.ds(xbase, _X_PER_W)])
        pltpu.sync_copy(rel_hbm.at[pl.ds(rbase, _REL_PER_W)], relbuf)
        pltpu.sync_copy(relbuf, rel_out_hbm.at[pl.ds(rbase, _REL_PER_W)])

    return sc_copy


_sc_copy = _make_sc_copy()


def kernel(x, edge_index, edge_type, rel_embed):
    x_out, rel_out = _sc_copy(x.reshape(-1), rel_embed.reshape(-1))
    return (x_out.reshape(x.shape), rel_out.reshape(rel_embed.shape))


# speedup vs baseline: 1.1266x; 1.1266x over previous
import functools

import jax
import jax.numpy as jnp
from jax import lax
from jax.experimental import pallas as pl
from jax.experimental.pallas import tpu as pltpu
from jax.experimental.pallas import tpu_sc as plsc

_info = plsc.get_sparse_core_info()
_NC, _NS = _info.num_cores, _info.num_subcores
_NW = _NC * _NS

_REL_FLAT = 500 * 128
_REL_PER_W = _REL_FLAT // _NW

_BLOCK_ROWS = 5000


def _make_sc_rel_copy():
    mesh = plsc.VectorSubcoreMesh(core_axis_name="c", subcore_axis_name="s")

    @functools.partial(
        pl.kernel,
        mesh=mesh,
        out_type=jax.ShapeDtypeStruct((_REL_FLAT,), jnp.float32),
        scratch_types=[pltpu.VMEM((_REL_PER_W,), jnp.float32)],
    )
    def sc_copy(rel_hbm, rel_out_hbm, relbuf):
        wid = lax.axis_index("s") * _NC + lax.axis_index("c")
        rbase = wid * _REL_PER_W
        pltpu.sync_copy(rel_hbm.at[pl.ds(rbase, _REL_PER_W)], relbuf)
        pltpu.sync_copy(relbuf, rel_out_hbm.at[pl.ds(rbase, _REL_PER_W)])

    return sc_copy


_sc_rel_copy = _make_sc_rel_copy()


def _x_copy(x_ref, x_out_ref):
    x_out_ref[...] = x_ref[...]


def kernel(x, edge_index, edge_type, rel_embed):
    n, d = x.shape
    rel_out = _sc_rel_copy(rel_embed.reshape(-1))
    x_out = pl.pallas_call(
        _x_copy,
        grid=(n // _BLOCK_ROWS,),
        in_specs=[pl.BlockSpec((_BLOCK_ROWS, d), lambda i: (i, 0))],
        out_specs=pl.BlockSpec((_BLOCK_ROWS, d), lambda i: (i, 0)),
        out_shape=jax.ShapeDtypeStruct(x.shape, x.dtype),
        compiler_params=pltpu.CompilerParams(
            dimension_semantics=("arbitrary",),
        ),
    )(x)
    return (x_out, rel_out.reshape(rel_embed.shape))


# speedup vs baseline: 5.0253x; 4.4606x over previous
import jax
from jax.experimental import pallas as pl
from jax.experimental.pallas import tpu as pltpu

_BLOCK_ROWS = 5000


def _copy_both(x_ref, rel_ref, x_out_ref, rel_out_ref):
    x_out_ref[...] = x_ref[...]
    rel_out_ref[...] = rel_ref[...]


def kernel(x, edge_index, edge_type, rel_embed):
    n, d = x.shape
    r, _ = rel_embed.shape
    x_out, rel_out = pl.pallas_call(
        _copy_both,
        grid=(n // _BLOCK_ROWS,),
        in_specs=[
            pl.BlockSpec((_BLOCK_ROWS, d), lambda i: (i, 0)),
            pl.BlockSpec((r, d), lambda i: (0, 0)),
        ],
        out_specs=[
            pl.BlockSpec((_BLOCK_ROWS, d), lambda i: (i, 0)),
            pl.BlockSpec((r, d), lambda i: (0, 0)),
        ],
        out_shape=[
            jax.ShapeDtypeStruct(x.shape, x.dtype),
            jax.ShapeDtypeStruct(rel_embed.shape, rel_embed.dtype),
        ],
        compiler_params=pltpu.CompilerParams(
            dimension_semantics=("arbitrary",),
        ),
    )(x, rel_embed)
    return (x_out, rel_out)
